# baseline (speedup 1.0000x reference)
import functools

import jax
import jax.numpy as jnp
from jax import lax
from jax.experimental import pallas as pl
from jax.experimental.pallas import tpu as pltpu
from jax.experimental.pallas import tpu_sc as plsc

NC = 2
NS = 16
L = 16
NW = NC * NS

SUB = 8
SPAN = 512


def _sweep_body(b_per_w, d, nrows,
                sorted_hbm, tableT_hbm, g_hbm,
                sid_v, bufs0, bufs1, bufs2, bufs3,
                bufs4, bufs5, bufs6, bufs7, tail_v, rowbuf,
                sem0, sem1, sem2, sem3, sem4, sem5, sem6, sem7):
    NBUF = 8
    tail_start = (nrows // 128) * 128
    tail_w = nrows - tail_start
    maxblk = tail_start // 128 - 1

    bufs = [bufs0, bufs1, bufs2, bufs3, bufs4, bufs5, bufs6, bufs7]
    sems = [sem0, sem1, sem2, sem3, sem4, sem5, sem6, sem7]

    wid = lax.axis_index("s") * NC + lax.axis_index("c")
    base = wid * b_per_w

    pltpu.sync_copy(sorted_hbm.at[pl.ds(base, b_per_w)], sid_v)
    if tail_w:
        pltpu.sync_copy(tableT_hbm.at[:, pl.ds(tail_start, tail_w)], tail_v)

    lanes = lax.iota(jnp.int32, L)

    def fetch(blk, q):
        start = jnp.minimum(blk, maxblk) * 128
        return pltpu.async_copy(tableT_hbm.at[:, pl.ds(start, 128)],
                                bufs[q], sems[q])

    def harvest(blk, q, v, rows_b, rows_s):
        start = jnp.minimum(blk, maxblk) * 128
        idx = v - start
        active = (idx >= 0) & (idx < 128)
        idxc = jnp.clip(idx, 0, 127)
        for c in range(d):
            cvec = jnp.full((L,), c, jnp.int32)
            vals = plsc.load_gather(bufs[q], [cvec, idxc])
            plsc.store_scatter(rowbuf, [rows_b, rows_s, cvec], vals,
                               mask=active)

    def group(gg, carry, h):
        v = sid_v[pl.ds(gg * L, L)]
        b0 = v[0] // 128
        nblk = v[15] // 128 - b0 + 1
        nstep = (nblk + NBUF - 1) // NBUF
        rows = (gg - h * (b_per_w // L // 2)) * L + lanes
        rows_b = rows // SUB
        rows_s = lax.rem(rows, SUB)

        for q in range(NBUF - 1):
            fetch(b0 + q, q)

        def step(s, c2):
            i0 = s * NBUF
            for q in range(NBUF):
                fetch(b0 + i0 + q + NBUF - 1, (q + NBUF - 1) % NBUF)
                pltpu.make_async_copy(tableT_hbm.at[:, pl.ds(0, 128)],
                                      bufs[q], sems[q]).wait()
                harvest(b0 + i0 + q, q, v, rows_b, rows_s)
            return c2

        lax.fori_loop(0, nstep, step, 0)

        for q in range(NBUF - 1):
            pltpu.make_async_copy(tableT_hbm.at[:, pl.ds(0, 128)],
                                  bufs[q], sems[q]).wait()

        if tail_w:
            @pl.when(v[15] >= tail_start)
            def _():
                idx_t = v - tail_start
                active_t = idx_t >= 0
                idxc_t = jnp.clip(idx_t, 0, tail_w - 1)
                for c in range(d):
                    cvec = jnp.full((L,), c, jnp.int32)
                    vals = plsc.load_gather(tail_v, [cvec, idxc_t])
                    plsc.store_scatter(rowbuf, [rows_b, rows_s, cvec], vals,
                                       mask=active_t)
        return carry

    ngrp = b_per_w // L
    for h in range(2):
        lax.fori_loop(h * (ngrp // 2), (h + 1) * (ngrp // 2),
                      functools.partial(group, h=h), 0)
        hbase = base + h * (b_per_w // 2)
        pltpu.sync_copy(rowbuf,
                        g_hbm.at[pl.ds(hbase // SUB, b_per_w // 2 // SUB)])


def _lerp_body(b_per_w, d,
               short_hbm, inv_hbm, g_hbm, alpha_hbm, out_hbm,
               inv_v, rows_v, short_v, alpha_v, sem, ssem):
    half = b_per_w // 2
    wid = lax.axis_index("s") * NC + lax.axis_index("c")
    base = wid * b_per_w

    pltpu.sync_copy(inv_hbm.at[pl.ds(base, b_per_w)], inv_v)
    pltpu.sync_copy(alpha_hbm, alpha_v)

    a = 1.0 / (1.0 + jnp.exp(-alpha_v[...]))
    om_a = 1.0 - a

    for h in range(2):
        hbase = base + h * half

        def issue(g, carry, h=h):
            v = inv_v[pl.ds(h * half + g * L, L)]
            for l in range(L):
                sp = v[l]
                spb = sp // SUB
                spr = lax.rem(sp, SUB)
                jb = (g * L + l) // SUB
                pltpu.async_copy(g_hbm.at[spb, pl.ds(spr, 1)],
                                 rows_v.at[jb, pl.ds(l % SUB, 1)], sem)
            return carry

        lax.fori_loop(0, half // L, issue, 0)

        c0 = pltpu.async_copy(
            short_hbm.at[pl.ds(hbase // SUB, half // SUB)], short_v, ssem)
        pltpu.make_async_copy(out_hbm.at[pl.ds(0, half // SUB)],
                              rows_v, sem).wait()
        c0.wait()

        def body(j, carry):
            jb = j // SUB
            js = lax.rem(j, SUB)
            for dj in range(d // L):
                sl = pl.ds(dj * L, L)
                r = rows_v[jb, js, sl]
                s = short_v[jb, js, sl]
                short_v[jb, js, sl] = a * s + om_a * r
            return carry

        lax.fori_loop(0, half, body, 0, unroll=2)

        pltpu.sync_copy(short_v,
                        out_hbm.at[pl.ds(hbase // SUB, half // SUB)])


def kernel(short_term_interest, user_ids, long_term_emb, alpha):
    b, d = short_term_interest.shape
    nrows = long_term_emb.shape[0]
    b_per_w = b // NW

    ids = user_ids.astype(jnp.int32)
    order = jnp.argsort(ids).astype(jnp.int32)
    sorted_ids = jnp.take(ids, order, axis=0)
    inv = jnp.zeros((b,), jnp.int32).at[order].set(
        jnp.arange(b, dtype=jnp.int32))
    alpha_vec = jnp.broadcast_to(jnp.asarray(alpha, jnp.float32).reshape(()), (L,))
    tableT = jnp.transpose(long_term_emb)
    short_t = short_term_interest.reshape(b // SUB, SUB, d)
    tail_w = nrows - (nrows // 128) * 128

    mesh = plsc.VectorSubcoreMesh(core_axis_name="c", subcore_axis_name="s",
                                  num_cores=NC, num_subcores=NS)

    sweep = functools.partial(
        pl.kernel,
        out_type=jax.ShapeDtypeStruct((b // SUB, SUB, d), jnp.float32),
        mesh=mesh,
        scratch_types=[
            pltpu.VMEM((b_per_w,), jnp.int32),
            pltpu.VMEM((d, 128), jnp.float32),
            pltpu.VMEM((d, 128), jnp.float32),
            pltpu.VMEM((d, 128), jnp.float32),
            pltpu.VMEM((d, 128), jnp.float32),
            pltpu.VMEM((d, 128), jnp.float32),
            pltpu.VMEM((d, 128), jnp.float32),
            pltpu.VMEM((d, 128), jnp.float32),
            pltpu.VMEM((d, 128), jnp.float32),
            pltpu.VMEM((d, max(tail_w, 1)), jnp.float32),
            pltpu.VMEM((b_per_w // 2 // SUB, SUB, d), jnp.float32),
            pltpu.SemaphoreType.DMA,
            pltpu.SemaphoreType.DMA,
            pltpu.SemaphoreType.DMA,
            pltpu.SemaphoreType.DMA,
            pltpu.SemaphoreType.DMA,
            pltpu.SemaphoreType.DMA,
            pltpu.SemaphoreType.DMA,
            pltpu.SemaphoreType.DMA,
        ],
        compiler_params=pltpu.CompilerParams(needs_layout_passes=False),
    )(functools.partial(_sweep_body, b_per_w, d, nrows))
    gathered = sweep(sorted_ids, tableT)

    lerp = functools.partial(
        pl.kernel,
        out_type=jax.ShapeDtypeStruct((b // SUB, SUB, d), jnp.float32),
        mesh=mesh,
        scratch_types=[
            pltpu.VMEM((b_per_w,), jnp.int32),
            pltpu.VMEM((b_per_w // 2 // SUB, SUB, d), jnp.float32),
            pltpu.VMEM((b_per_w // 2 // SUB, SUB, d), jnp.float32),
            pltpu.VMEM((L,), jnp.float32),
            pltpu.SemaphoreType.DMA,
            pltpu.SemaphoreType.DMA,
        ],
    )(functools.partial(_lerp_body, b_per_w, d))
    out_t = lerp(short_t, inv, gathered, alpha_vec)
    return out_t.reshape(b, d)


# speedup vs baseline: 1.5962x; 1.5962x over previous
import functools

import jax
import jax.numpy as jnp
from jax import lax
from jax.experimental import pallas as pl
from jax.experimental.pallas import tpu as pltpu
from jax.experimental.pallas import tpu_sc as plsc

NC = 2
NS = 16
L = 16
NW = NC * NS

SUB = 8
SPAN = 512


def _sweep_body(b_per_w, d, nrows,
                sorted_hbm, tableT_hbm, g_hbm,
                sid_v, wide, tail_v, rowbuf, sem):
    NBLK = 8
    tail_start = (nrows // 128) * 128
    tail_w = nrows - tail_start
    maxblk = tail_start // 128 - 1

    wid = lax.axis_index("s") * NC + lax.axis_index("c")
    base = wid * b_per_w

    pltpu.sync_copy(sorted_hbm.at[pl.ds(base, b_per_w)], sid_v)
    if tail_w:
        pltpu.sync_copy(tableT_hbm.at[:, pl.ds(tail_start, tail_w)], tail_v)

    lanes = lax.iota(jnp.int32, L)

    def group(gg, carry, h):
        v = sid_v[pl.ds(gg * L, L)]
        b0 = jnp.minimum(v[0] // 128, maxblk)
        b1 = jnp.minimum(v[15] // 128, maxblk)
        nblk = b1 - b0 + 1
        npass = (nblk + NBLK - 1) // NBLK
        rows = (gg - h * (b_per_w // L // 2)) * L + lanes
        rows_b = rows // SUB
        rows_s = lax.rem(rows, SUB)

        def gpass(t, c2):
            blk0 = b0 + t * NBLK
            nf = jnp.minimum(nblk - t * NBLK, NBLK)

            def fire(qq, c3):
                pltpu.async_copy(
                    tableT_hbm.at[:, pl.ds((blk0 + qq) * 128, 128)],
                    wide.at[:, pl.ds(qq * 128, 128)], sem)
                return c3

            lax.fori_loop(0, nf, fire, 0)

            def drain(qq, c3):
                pltpu.make_async_copy(
                    tableT_hbm.at[:, pl.ds(0, 128)],
                    wide.at[:, pl.ds(qq * 128, 128)], sem).wait()
                return c3

            lax.fori_loop(0, nf, drain, 0)

            idx = v - blk0 * 128
            active = (idx >= 0) & (idx < NBLK * 128)
            idxc = jnp.clip(idx, 0, NBLK * 128 - 1)
            for c in range(d):
                cvec = jnp.full((L,), c, jnp.int32)
                vals = plsc.load_gather(wide, [cvec, idxc])
                plsc.store_scatter(rowbuf, [rows_b, rows_s, cvec], vals,
                                   mask=active)
            return c2

        lax.fori_loop(0, npass, gpass, 0)

        if tail_w:
            @pl.when(v[15] >= tail_start)
            def _():
                idx_t = v - tail_start
                active_t = idx_t >= 0
                idxc_t = jnp.clip(idx_t, 0, tail_w - 1)
                for c in range(d):
                    cvec = jnp.full((L,), c, jnp.int32)
                    vals = plsc.load_gather(tail_v, [cvec, idxc_t])
                    plsc.store_scatter(rowbuf, [rows_b, rows_s, cvec], vals,
                                       mask=active_t)
        return carry

    ngrp = b_per_w // L
    for h in range(2):
        lax.fori_loop(h * (ngrp // 2), (h + 1) * (ngrp // 2),
                      functools.partial(group, h=h), 0)
        hbase = base + h * (b_per_w // 2)
        pltpu.sync_copy(rowbuf,
                        g_hbm.at[pl.ds(hbase // SUB, b_per_w // 2 // SUB)])


def _lerp_body(b_per_w, d,
               short_hbm, inv_hbm, g_hbm, alpha_hbm, out_hbm,
               inv_v, rows_v, short_v, alpha_v, sem, ssem):
    half = b_per_w // 2
    wid = lax.axis_index("s") * NC + lax.axis_index("c")
    base = wid * b_per_w

    pltpu.sync_copy(inv_hbm.at[pl.ds(base, b_per_w)], inv_v)
    pltpu.sync_copy(alpha_hbm, alpha_v)

    a = 1.0 / (1.0 + jnp.exp(-alpha_v[...]))
    om_a = 1.0 - a

    for h in range(2):
        hbase = base + h * half

        def issue(g, carry, h=h):
            v = inv_v[pl.ds(h * half + g * L, L)]
            for l in range(L):
                sp = v[l]
                spb = sp // SUB
                spr = lax.rem(sp, SUB)
                jb = (g * L + l) // SUB
                pltpu.async_copy(g_hbm.at[spb, pl.ds(spr, 1)],
                                 rows_v.at[jb, pl.ds(l % SUB, 1)], sem)
            return carry

        lax.fori_loop(0, half // L, issue, 0)

        c0 = pltpu.async_copy(
            short_hbm.at[pl.ds(hbase // SUB, half // SUB)], short_v, ssem)
        pltpu.make_async_copy(out_hbm.at[pl.ds(0, half // SUB)],
                              rows_v, sem).wait()
        c0.wait()

        def body(j, carry):
            jb = j // SUB
            js = lax.rem(j, SUB)
            for dj in range(d // L):
                sl = pl.ds(dj * L, L)
                r = rows_v[jb, js, sl]
                s = short_v[jb, js, sl]
                short_v[jb, js, sl] = a * s + om_a * r
            return carry

        lax.fori_loop(0, half, body, 0, unroll=2)

        pltpu.sync_copy(short_v,
                        out_hbm.at[pl.ds(hbase // SUB, half // SUB)])


def kernel(short_term_interest, user_ids, long_term_emb, alpha):
    b, d = short_term_interest.shape
    nrows = long_term_emb.shape[0]
    b_per_w = b // NW

    ids = user_ids.astype(jnp.int32)
    order = jnp.argsort(ids).astype(jnp.int32)
    sorted_ids = jnp.take(ids, order, axis=0)
    inv = jnp.zeros((b,), jnp.int32).at[order].set(
        jnp.arange(b, dtype=jnp.int32))
    alpha_vec = jnp.broadcast_to(jnp.asarray(alpha, jnp.float32).reshape(()), (L,))
    tableT = jnp.transpose(long_term_emb)
    short_t = short_term_interest.reshape(b // SUB, SUB, d)
    tail_w = nrows - (nrows // 128) * 128

    mesh = plsc.VectorSubcoreMesh(core_axis_name="c", subcore_axis_name="s",
                                  num_cores=NC, num_subcores=NS)

    sweep = functools.partial(
        pl.kernel,
        out_type=jax.ShapeDtypeStruct((b // SUB, SUB, d), jnp.float32),
        mesh=mesh,
        scratch_types=[
            pltpu.VMEM((b_per_w,), jnp.int32),
            pltpu.VMEM((d, 1024), jnp.float32),
            pltpu.VMEM((d, max(tail_w, 1)), jnp.float32),
            pltpu.VMEM((b_per_w // 2 // SUB, SUB, d), jnp.float32),
            pltpu.SemaphoreType.DMA,
        ],
        compiler_params=pltpu.CompilerParams(needs_layout_passes=False),
    )(functools.partial(_sweep_body, b_per_w, d, nrows))
    gathered = sweep(sorted_ids, tableT)

    lerp = functools.partial(
        pl.kernel,
        out_type=jax.ShapeDtypeStruct((b // SUB, SUB, d), jnp.float32),
        mesh=mesh,
        scratch_types=[
            pltpu.VMEM((b_per_w,), jnp.int32),
            pltpu.VMEM((b_per_w // 2 // SUB, SUB, d), jnp.float32),
            pltpu.VMEM((b_per_w // 2 // SUB, SUB, d), jnp.float32),
            pltpu.VMEM((L,), jnp.float32),
            pltpu.SemaphoreType.DMA,
            pltpu.SemaphoreType.DMA,
        ],
    )(functools.partial(_lerp_body, b_per_w, d))
    out_t = lerp(short_t, inv, gathered, alpha_vec)
    return out_t.reshape(b, d)


# speedup vs baseline: 1.6455x; 1.0309x over previous
import functools

import jax
import jax.numpy as jnp
from jax import lax
from jax.experimental import pallas as pl
from jax.experimental.pallas import tpu as pltpu
from jax.experimental.pallas import tpu_sc as plsc

NC = 2
NS = 16
L = 16
NW = NC * NS

SUB = 8
SPAN = 512


def _sweep_body(b_per_w, d, nrows,
                sorted_hbm, tableT_hbm, g_hbm,
                sid_v, wide, tail_v, rowbuf, sem):
    NBLK = 10
    tail_start = (nrows // 128) * 128
    tail_w = nrows - tail_start
    maxblk = tail_start // 128 - 1

    wid = lax.axis_index("s") * NC + lax.axis_index("c")
    base = wid * b_per_w

    pltpu.sync_copy(sorted_hbm.at[pl.ds(base, b_per_w)], sid_v)
    if tail_w:
        pltpu.sync_copy(tableT_hbm.at[:, pl.ds(tail_start, tail_w)], tail_v)

    lanes = lax.iota(jnp.int32, L)

    def group(gg, carry, h):
        v = sid_v[pl.ds(gg * L, L)]
        b0 = jnp.minimum(v[0] // 128, maxblk)
        b1 = jnp.minimum(v[15] // 128, maxblk)
        nblk = b1 - b0 + 1
        npass = (nblk + NBLK - 1) // NBLK
        rows = (gg - h * (b_per_w // L // 2)) * L + lanes
        rows_b = rows // SUB
        rows_s = lax.rem(rows, SUB)

        def gpass(t, c2):
            blk0 = b0 + t * NBLK
            nf = jnp.minimum(nblk - t * NBLK, NBLK)

            def fire(qq, c3):
                pltpu.async_copy(
                    tableT_hbm.at[:, pl.ds((blk0 + qq) * 128, 128)],
                    wide.at[:, pl.ds(qq * 128, 128)], sem)
                return c3

            lax.fori_loop(0, nf, fire, 0)

            def drain(qq, c3):
                pltpu.make_async_copy(
                    tableT_hbm.at[:, pl.ds(0, 128)],
                    wide.at[:, pl.ds(qq * 128, 128)], sem).wait()
                return c3

            lax.fori_loop(0, nf, drain, 0)

            idx = v - blk0 * 128
            active = (idx >= 0) & (idx < NBLK * 128)
            idxc = jnp.clip(idx, 0, NBLK * 128 - 1)
            for c in range(d):
                cvec = jnp.full((L,), c, jnp.int32)
                vals = plsc.load_gather(wide, [cvec, idxc])
                plsc.store_scatter(rowbuf, [rows_b, rows_s, cvec], vals,
                                   mask=active)
            return c2

        lax.fori_loop(0, npass, gpass, 0)

        if tail_w:
            @pl.when(v[15] >= tail_start)
            def _():
                idx_t = v - tail_start
                active_t = idx_t >= 0
                idxc_t = jnp.clip(idx_t, 0, tail_w - 1)
                for c in range(d):
                    cvec = jnp.full((L,), c, jnp.int32)
                    vals = plsc.load_gather(tail_v, [cvec, idxc_t])
                    plsc.store_scatter(rowbuf, [rows_b, rows_s, cvec], vals,
                                       mask=active_t)
        return carry

    ngrp = b_per_w // L
    for h in range(2):
        lax.fori_loop(h * (ngrp // 2), (h + 1) * (ngrp // 2),
                      functools.partial(group, h=h), 0)
        hbase = base + h * (b_per_w // 2)
        pltpu.sync_copy(rowbuf,
                        g_hbm.at[pl.ds(hbase // SUB, b_per_w // 2 // SUB)])


def _lerp_body(b_per_w, d,
               short_hbm, inv_hbm, g_hbm, alpha_hbm, out_hbm,
               inv_v, rows_v, short_v, alpha_v, sem, ssem):
    half = b_per_w // 2
    wid = lax.axis_index("s") * NC + lax.axis_index("c")
    base = wid * b_per_w

    pltpu.sync_copy(inv_hbm.at[pl.ds(base, b_per_w)], inv_v)
    pltpu.sync_copy(alpha_hbm, alpha_v)

    a = 1.0 / (1.0 + jnp.exp(-alpha_v[...]))
    om_a = 1.0 - a

    for h in range(2):
        hbase = base + h * half

        def issue(g, carry, h=h):
            v = inv_v[pl.ds(h * half + g * L, L)]
            for l in range(L):
                sp = v[l]
                spb = sp // SUB
                spr = lax.rem(sp, SUB)
                jb = (g * L + l) // SUB
                pltpu.async_copy(g_hbm.at[spb, pl.ds(spr, 1)],
                                 rows_v.at[jb, pl.ds(l % SUB, 1)], sem)
            return carry

        lax.fori_loop(0, half // L, issue, 0)

        c0 = pltpu.async_copy(
            short_hbm.at[pl.ds(hbase // SUB, half // SUB)], short_v, ssem)
        pltpu.make_async_copy(out_hbm.at[pl.ds(0, half // SUB)],
                              rows_v, sem).wait()
        c0.wait()

        def body(j, carry):
            jb = j // SUB
            js = lax.rem(j, SUB)
            for dj in range(d // L):
                sl = pl.ds(dj * L, L)
                r = rows_v[jb, js, sl]
                s = short_v[jb, js, sl]
                short_v[jb, js, sl] = a * s + om_a * r
            return carry

        lax.fori_loop(0, half, body, 0, unroll=2)

        pltpu.sync_copy(short_v,
                        out_hbm.at[pl.ds(hbase // SUB, half // SUB)])


def kernel(short_term_interest, user_ids, long_term_emb, alpha):
    b, d = short_term_interest.shape
    nrows = long_term_emb.shape[0]
    b_per_w = b // NW

    ids = user_ids.astype(jnp.int32)
    order = jnp.argsort(ids).astype(jnp.int32)
    sorted_ids = jnp.take(ids, order, axis=0)
    inv = jnp.zeros((b,), jnp.int32).at[order].set(
        jnp.arange(b, dtype=jnp.int32))
    alpha_vec = jnp.broadcast_to(jnp.asarray(alpha, jnp.float32).reshape(()), (L,))
    tableT = jnp.transpose(long_term_emb)
    short_t = short_term_interest.reshape(b // SUB, SUB, d)
    tail_w = nrows - (nrows // 128) * 128

    mesh = plsc.VectorSubcoreMesh(core_axis_name="c", subcore_axis_name="s",
                                  num_cores=NC, num_subcores=NS)

    sweep = functools.partial(
        pl.kernel,
        out_type=jax.ShapeDtypeStruct((b // SUB, SUB, d), jnp.float32),
        mesh=mesh,
        scratch_types=[
            pltpu.VMEM((b_per_w,), jnp.int32),
            pltpu.VMEM((d, 1280), jnp.float32),
            pltpu.VMEM((d, max(tail_w, 1)), jnp.float32),
            pltpu.VMEM((b_per_w // 2 // SUB, SUB, d), jnp.float32),
            pltpu.SemaphoreType.DMA,
        ],
        compiler_params=pltpu.CompilerParams(needs_layout_passes=False),
    )(functools.partial(_sweep_body, b_per_w, d, nrows))
    gathered = sweep(sorted_ids, tableT)

    lerp = functools.partial(
        pl.kernel,
        out_type=jax.ShapeDtypeStruct((b // SUB, SUB, d), jnp.float32),
        mesh=mesh,
        scratch_types=[
            pltpu.VMEM((b_per_w,), jnp.int32),
            pltpu.VMEM((b_per_w // 2 // SUB, SUB, d), jnp.float32),
            pltpu.VMEM((b_per_w // 2 // SUB, SUB, d), jnp.float32),
            pltpu.VMEM((L,), jnp.float32),
            pltpu.SemaphoreType.DMA,
            pltpu.SemaphoreType.DMA,
        ],
    )(functools.partial(_lerp_body, b_per_w, d))
    out_t = lerp(short_t, inv, gathered, alpha_vec)
    return out_t.reshape(b, d)


# speedup vs baseline: 1.6880x; 1.0258x over previous
import functools

import jax
import jax.numpy as jnp
from jax import lax
from jax.experimental import pallas as pl
from jax.experimental.pallas import tpu as pltpu
from jax.experimental.pallas import tpu_sc as plsc

NC = 2
NS = 16
L = 16
NW = NC * NS

SUB = 8
SPAN = 512


def _sweep_body(b_per_w, d, nrows,
                sorted_hbm, tableT_hbm, g_hbm,
                sid_v, wide, tail_v, rowbuf, sem):
    NBLK = 10
    tail_start = (nrows // 128) * 128
    tail_w = nrows - tail_start
    maxblk = tail_start // 128 - 1

    wid = lax.axis_index("s") * NC + lax.axis_index("c")
    base = wid * b_per_w

    pltpu.sync_copy(sorted_hbm.at[pl.ds(base, b_per_w)], sid_v)
    if tail_w:
        pltpu.sync_copy(tableT_hbm.at[:, pl.ds(tail_start, tail_w)], tail_v)

    lanes = lax.iota(jnp.int32, L)

    def group(gg, carry, h):
        v = sid_v[pl.ds(gg * L, L)]
        b0 = jnp.minimum(v[0] // 128, maxblk)
        b1 = jnp.minimum(v[15] // 128, maxblk)
        nblk = b1 - b0 + 1
        npass = (nblk + NBLK - 1) // NBLK
        rows = (gg - h * (b_per_w // L // 2)) * L + lanes
        rows_b = rows // SUB
        rows_s = lax.rem(rows, SUB)

        def gpass(t, c2):
            blk0 = b0 + t * NBLK
            nf = jnp.minimum(nblk - t * NBLK, NBLK)

            def fire(qq, c3):
                pltpu.async_copy(
                    tableT_hbm.at[:, pl.ds((blk0 + qq) * 128, 128)],
                    wide.at[:, pl.ds(qq * 128, 128)], sem)
                return c3

            lax.fori_loop(0, nf, fire, 0)

            def drain(qq, c3):
                pltpu.make_async_copy(
                    tableT_hbm.at[:, pl.ds(0, 128)],
                    wide.at[:, pl.ds(qq * 128, 128)], sem).wait()
                return c3

            lax.fori_loop(0, nf, drain, 0)

            idx = v - blk0 * 128
            active = (idx >= 0) & (idx < NBLK * 128)
            idxc = jnp.clip(idx, 0, NBLK * 128 - 1)
            for c in range(d):
                cvec = jnp.full((L,), c, jnp.int32)
                vals = plsc.load_gather(wide, [cvec, idxc])
                plsc.store_scatter(rowbuf, [rows_b, rows_s, cvec], vals,
                                   mask=active)
            return c2

        lax.fori_loop(0, npass, gpass, 0)

        if tail_w:
            @pl.when(v[15] >= tail_start)
            def _():
                idx_t = v - tail_start
                active_t = idx_t >= 0
                idxc_t = jnp.clip(idx_t, 0, tail_w - 1)
                for c in range(d):
                    cvec = jnp.full((L,), c, jnp.int32)
                    vals = plsc.load_gather(tail_v, [cvec, idxc_t])
                    plsc.store_scatter(rowbuf, [rows_b, rows_s, cvec], vals,
                                       mask=active_t)
        return carry

    ngrp = b_per_w // L
    for h in range(2):
        lax.fori_loop(h * (ngrp // 2), (h + 1) * (ngrp // 2),
                      functools.partial(group, h=h), 0)
        hbase = base + h * (b_per_w // 2)
        pltpu.sync_copy(rowbuf,
                        g_hbm.at[pl.ds(hbase // SUB, b_per_w // 2 // SUB)])


def _lerp_body(b_per_w, d,
               short_hbm, inv_hbm, g_hbm, alpha_hbm, out_hbm,
               inv_v, rows_v, short_v, alpha_v, sem, ssem):
    half = b_per_w // 2
    wid = lax.axis_index("s") * NC + lax.axis_index("c")
    base = wid * b_per_w

    pltpu.sync_copy(inv_hbm.at[pl.ds(base, b_per_w)], inv_v)
    pltpu.sync_copy(alpha_hbm, alpha_v)

    def issue(g, carry):
        v = inv_v[pl.ds(g * L, L)]
        for l in range(L):
            sp = v[l]
            spb = sp // SUB
            spr = lax.rem(sp, SUB)
            jb = (g * L + l) // SUB
            pltpu.async_copy(g_hbm.at[spb, pl.ds(spr, 1)],
                             rows_v.at[jb, pl.ds(l % SUB, 1)], sem)
        return carry

    lax.fori_loop(0, b_per_w // L, issue, 0)

    c0 = pltpu.async_copy(
        short_hbm.at[pl.ds(base // SUB, half // SUB)], short_v, ssem)
    pltpu.make_async_copy(out_hbm.at[pl.ds(0, b_per_w // SUB)],
                          rows_v, sem).wait()

    a = 1.0 / (1.0 + jnp.exp(-alpha_v[...]))
    om_a = 1.0 - a

    c0.wait()
    for h in range(2):
        hbase = base + h * half

        def body(j, carry, h=h):
            jb = h * (half // SUB) + j // SUB
            sb = j // SUB
            js = lax.rem(j, SUB)
            for dj in range(d // L):
                sl = pl.ds(dj * L, L)
                r = rows_v[jb, js, sl]
                s = short_v[sb, js, sl]
                short_v[sb, js, sl] = a * s + om_a * r
            return carry

        lax.fori_loop(0, half, body, 0, unroll=2)

        pltpu.sync_copy(short_v,
                        out_hbm.at[pl.ds(hbase // SUB, half // SUB)])
        if h == 0:
            pltpu.sync_copy(
                short_hbm.at[pl.ds((base + half) // SUB, half // SUB)],
                short_v)


def kernel(short_term_interest, user_ids, long_term_emb, alpha):
    b, d = short_term_interest.shape
    nrows = long_term_emb.shape[0]
    b_per_w = b // NW

    ids = user_ids.astype(jnp.int32)
    order = jnp.argsort(ids).astype(jnp.int32)
    sorted_ids = jnp.take(ids, order, axis=0)
    inv = jnp.zeros((b,), jnp.int32).at[order].set(
        jnp.arange(b, dtype=jnp.int32))
    alpha_vec = jnp.broadcast_to(jnp.asarray(alpha, jnp.float32).reshape(()), (L,))
    tableT = jnp.transpose(long_term_emb)
    short_t = short_term_interest.reshape(b // SUB, SUB, d)
    tail_w = nrows - (nrows // 128) * 128

    mesh = plsc.VectorSubcoreMesh(core_axis_name="c", subcore_axis_name="s",
                                  num_cores=NC, num_subcores=NS)

    sweep = functools.partial(
        pl.kernel,
        out_type=jax.ShapeDtypeStruct((b // SUB, SUB, d), jnp.float32),
        mesh=mesh,
        scratch_types=[
            pltpu.VMEM((b_per_w,), jnp.int32),
            pltpu.VMEM((d, 1280), jnp.float32),
            pltpu.VMEM((d, max(tail_w, 1)), jnp.float32),
            pltpu.VMEM((b_per_w // 2 // SUB, SUB, d), jnp.float32),
            pltpu.SemaphoreType.DMA,
        ],
        compiler_params=pltpu.CompilerParams(needs_layout_passes=False),
    )(functools.partial(_sweep_body, b_per_w, d, nrows))
    gathered = sweep(sorted_ids, tableT)

    lerp = functools.partial(
        pl.kernel,
        out_type=jax.ShapeDtypeStruct((b // SUB, SUB, d), jnp.float32),
        mesh=mesh,
        scratch_types=[
            pltpu.VMEM((b_per_w,), jnp.int32),
            pltpu.VMEM((b_per_w // SUB, SUB, d), jnp.float32),
            pltpu.VMEM((b_per_w // 2 // SUB, SUB, d), jnp.float32),
            pltpu.VMEM((L,), jnp.float32),
            pltpu.SemaphoreType.DMA,
            pltpu.SemaphoreType.DMA,
        ],
    )(functools.partial(_lerp_body, b_per_w, d))
    out_t = lerp(short_t, inv, gathered, alpha_vec)
    return out_t.reshape(b, d)
